# baseline (speedup 1.0000x reference)
import functools

import jax
import jax.numpy as jnp
from jax import lax
from jax.experimental import pallas as pl
from jax.experimental.pallas import tpu as pltpu
from jax.experimental.pallas import tpu_sc as plsc

N = 10000
E = 320000
D = 128
DE = 11
NC = 2
NS = 16
NW = NC * NS
EPW = E // NW
B = 80
NB = EPW // B
SB = 25
PW = 144
NP = 10240
RPT = NP // NS

_mesh = plsc.VectorSubcoreMesh(core_axis_name="c", subcore_axis_name="s")
_sc_params = pltpu.CompilerParams(needs_layout_passes=False,
                                  use_tc_tiling_on_sc=False)


def _proj_body(x_ref, wl_ref, wr_ref, xl_ref, xr_ref, xe_ref):
    x = x_ref[...]
    xl = jnp.dot(x, wl_ref[...], preferred_element_type=jnp.float32)
    xl_ref[...] = xl
    xr_ref[...] = jnp.dot(x, wr_ref[...], preferred_element_type=jnp.float32)
    xe_ref[...] = jnp.concatenate(
        [xl, jnp.ones((N, PW - D), jnp.float32)], axis=1)


def _proj(x, W_l, W_r):
    return pl.pallas_call(
        _proj_body,
        out_shape=(
            jax.ShapeDtypeStruct((N, D), jnp.float32),
            jax.ShapeDtypeStruct((N, D), jnp.float32),
            jax.ShapeDtypeStruct((N, PW), jnp.float32),
        ),
    )(x, W_l, W_r)


@functools.partial(
    pl.kernel,
    out_type=jax.ShapeDtypeStruct((E, D), jnp.float32),
    mesh=_mesh,
    scratch_types=[
        pltpu.VMEM((NB, B), jnp.int32),
        pltpu.VMEM((NB, B), jnp.int32),
        [pltpu.VMEM((B, D), jnp.float32)] * 3,
        [pltpu.VMEM((B, D), jnp.float32)] * 3,
        [pltpu.VMEM((B, D), jnp.float32)] * 3,
        [pltpu.SemaphoreType.DMA] * 3,
        [pltpu.SemaphoreType.DMA] * 3,
        [pltpu.SemaphoreType.DMA] * 3,
    ],
    compiler_params=_sc_params,
)
def _gather_add(xl_hbm, xr_hbm, src_hbm, dst_hbm, u_hbm,
                si_v, di_v, gl, gr, ob, gsl, gsr, ss):
    wid = lax.axis_index("s") * NC + lax.axis_index("c")
    pltpu.sync_copy(src_hbm.at[wid], si_v)
    pltpu.sync_copy(dst_hbm.at[wid], di_v)
    base = wid * EPW

    def fire(blk, k):
        pltpu.async_copy(xl_hbm.at[si_v.at[blk]], gl[k], gsl[k])
        pltpu.async_copy(xr_hbm.at[di_v.at[blk]], gr[k], gsr[k])

    def wait_g(blk, k):
        pltpu.make_async_copy(xl_hbm.at[si_v.at[blk]], gl[k], gsl[k]).wait()
        pltpu.make_async_copy(xr_hbm.at[di_v.at[blk]], gr[k], gsr[k]).wait()

    def compute(k):
        @plsc.parallel_loop(0, B, unroll=8)
        def _row(j):
            for cch in range(D // 16):
                sl = pl.ds(cch * 16, 16)
                ob[k][j, sl] = gl[k][j, sl] + gr[k][j, sl]

    def udst(blk):
        return u_hbm.at[pl.ds(base + blk * B, B)]

    def wait_s(blk, k):
        pltpu.make_async_copy(ob[k], udst(blk), ss[k]).wait()

    for k in range(3):
        fire(k, k)

    @pl.loop(0, NB - 2, step=3)
    def _trip(g):
        for k in range(3):
            b = g + k
            wait_g(b, k)

            @pl.when(g > 0)
            def _():
                wait_s(b - 3, k)

            compute(k)
            pltpu.async_copy(ob[k], udst(b), ss[k])
            if k < 2:
                fire(b + 3, k)
            else:
                @pl.when(g < NB - 5)
                def _():
                    fire(b + 3, k)

    for k, b in ((0, NB - 2), (1, NB - 1)):
        wait_g(b, k)
        wait_s(b - 3, k)
        compute(k)
        pltpu.async_copy(ob[k], udst(b), ss[k])
    wait_s(NB - 2, 0)
    wait_s(NB - 1, 1)
    wait_s(NB - 3, 2)


EB = 6400
GRID_A = E // EB


def _alpha_body(u_ref, ea_ref, we_ref, att_ref, alpha_ref, gmax_ref):
    e = lax.dot_general(ea_ref[...], we_ref[...], (((0,), (0,)), ((), ())),
                        preferred_element_type=jnp.float32)
    m = u_ref[...] + e
    m = jnp.where(m > 0, m, 0.2 * m)
    a = jnp.dot(m, att_ref[...], preferred_element_type=jnp.float32)[:, 0]
    alpha_ref[...] = a.reshape(1, EB // 128, 128)
    bm = jnp.max(a)

    @pl.when(pl.program_id(0) == 0)
    def _():
        gmax_ref[...] = jnp.full((1, 1), bm)

    @pl.when(pl.program_id(0) != 0)
    def _():
        gmax_ref[...] = jnp.maximum(gmax_ref[...], bm)


def _alpha(u, edge_attr, W_e, att):
    return pl.pallas_call(
        _alpha_body,
        grid=(GRID_A,),
        in_specs=[
            pl.BlockSpec((EB, D), lambda i: (i, 0)),
            pl.BlockSpec((DE, EB), lambda i: (0, i)),
            pl.BlockSpec((DE, D), lambda i: (0, 0)),
            pl.BlockSpec((D, 1), lambda i: (0, 0)),
        ],
        out_specs=[
            pl.BlockSpec((1, EB // 128, 128), lambda i: (i, 0, 0)),
            pl.BlockSpec((1, 1), lambda i: (0, 0)),
        ],
        out_shape=[
            jax.ShapeDtypeStruct((GRID_A, EB // 128, 128), jnp.float32),
            jax.ShapeDtypeStruct((1, 1), jnp.float32),
        ],
    )(u, edge_attr.T, W_e, att.reshape(D, 1))


@functools.partial(
    pl.kernel,
    out_type=jax.ShapeDtypeStruct((NC, NP, PW), jnp.float32),
    mesh=_mesh,
    scratch_types=[
        pltpu.VMEM((SB, B), jnp.int32),
        pltpu.VMEM((SB, B), jnp.int32),
        pltpu.VMEM((SB, B), jnp.float32),
        pltpu.VMEM((B, PW), jnp.float32),
        pltpu.VMEM((B, PW), jnp.float32),
        pltpu.VMEM((16,), jnp.float32),
        pltpu.VMEM_SHARED((NP, PW), jnp.float32),
        pltpu.SemaphoreType.DMA,
        pltpu.SemaphoreType.DMA,
        pltpu.SemaphoreType.DMA,
        pltpu.SemaphoreType.DMA,
    ],
    compiler_params=_sc_params,
)
def _scatter(xe_hbm, src_hbm, dst_hbm, alpha_hbm, gmax_hbm, zeros_hbm, part_hbm,
             si_v, di_v, al_v, gl0, gl1, gm_v, acc_sh, gs0, gs1, ss0, ss1):
    cid = lax.axis_index("c")
    sid = lax.axis_index("s")
    wid = sid * NC + cid
    pltpu.sync_copy(gmax_hbm, gm_v)
    pltpu.sync_copy(zeros_hbm.at[pl.ds(sid * RPT, RPT)],
                    acc_sh.at[pl.ds(sid * RPT, RPT)])
    plsc.subcore_barrier()

    def mult(blk, buf):
        blk16 = lax.broadcast(blk, (16,))

        @plsc.parallel_loop(0, B, unroll=8)
        def _row(j):
            j16 = lax.broadcast(j, (16,))
            eaj = plsc.load_gather(al_v, [blk16, j16])
            for cch in range(PW // 16):
                sl = pl.ds(cch * 16, 16)
                buf[j, sl] = buf[j, sl] * eaj

    def fire_g(blk, buf, sem):
        pltpu.async_copy(xe_hbm.at[si_v.at[blk]], buf, sem)

    def wait_g(blk, buf, sem):
        pltpu.make_async_copy(xe_hbm.at[si_v.at[blk]], buf, sem).wait()

    def fire_s(blk, buf, sem):
        pltpu.async_copy(buf, acc_sh.at[di_v.at[blk]], sem, add=True)

    def wait_s(buf, sem):
        pltpu.make_async_copy(buf, acc_sh.at[di_v.at[0]], sem).wait()

    @pl.loop(0, NB // SB)
    def _sblk(sb):
        @pl.when(sb > 0)
        def _():
            wait_s(gl1, ss1)

        pltpu.sync_copy(src_hbm.at[wid].at[pl.ds(sb * SB, SB)], si_v)
        pltpu.sync_copy(dst_hbm.at[wid].at[pl.ds(sb * SB, SB)], di_v)
        pltpu.sync_copy(alpha_hbm.at[wid].at[pl.ds(sb * SB, SB)], al_v)

        @pl.loop(0, SB)
        def _ea(r):
            for k in range(B // 16):
                sl = pl.ds(k * 16, 16)
                al_v[r, sl] = jnp.exp(al_v[r, sl] - gm_v[...])

        fire_g(0, gl0, gs0)

        @pl.loop(0, SB - 1, step=2)
        def _pair(g):
            @pl.when(g > 0)
            def _():
                wait_s(gl1, ss1)

            fire_g(g + 1, gl1, gs1)
            wait_g(g, gl0, gs0)
            mult(g, gl0)
            fire_s(g, gl0, ss0)
            wait_g(g + 1, gl1, gs1)
            mult(g + 1, gl1)
            fire_s(g + 1, gl1, ss1)
            wait_s(gl0, ss0)

            @pl.when(g + 2 < SB)
            def _():
                fire_g(g + 2, gl0, gs0)

        wait_g(SB - 1, gl0, gs0)
        mult(SB - 1, gl0)
        fire_s(SB - 1, gl0, ss0)
        wait_s(gl0, ss0)

    wait_s(gl1, ss1)
    plsc.subcore_barrier()
    st = sid * RPT
    pltpu.sync_copy(acc_sh.at[pl.ds(st, RPT)],
                    part_hbm.at[cid].at[pl.ds(st, RPT)])


FRB = 40
NF = NP // NW


@functools.partial(
    pl.kernel,
    out_type=jax.ShapeDtypeStruct((N, D), jnp.float32),
    mesh=_mesh,
    scratch_types=[
        pltpu.VMEM((FRB, PW), jnp.float32),
        pltpu.VMEM((FRB, PW), jnp.float32),
        pltpu.VMEM((FRB, D), jnp.float32),
        pltpu.SemaphoreType.DMA,
        pltpu.SemaphoreType.DMA,
    ],
    compiler_params=_sc_params,
)
def _finalize(part_hbm, out_hbm, p0_v, p1_v, o_v, s0, s1):
    wid = lax.axis_index("s") * NC + lax.axis_index("c")
    base = wid * NF

    @pl.loop(0, NF // FRB)
    def _blk(b):
        r0 = base + b * FRB

        @pl.when(r0 < N)
        def _():
            c0 = pltpu.async_copy(part_hbm.at[0].at[pl.ds(r0, FRB)], p0_v, s0)
            c1 = pltpu.async_copy(part_hbm.at[1].at[pl.ds(r0, FRB)], p1_v, s1)
            c0.wait()
            c1.wait()

            @plsc.parallel_loop(0, FRB, unroll=4)
            def _row(j):
                d16 = (p0_v[j, pl.ds(D, 16)] + p1_v[j, pl.ds(D, 16)]) + 1e-16
                for cch in range(D // 16):
                    sl = pl.ds(cch * 16, 16)
                    o_v[j, sl] = (p0_v[j, sl] + p1_v[j, sl]) / d16

            pltpu.sync_copy(o_v, out_hbm.at[pl.ds(r0, FRB)])


def kernel(x, edge_index, edge_attr, W_l, W_r, W_e, att):
    src3 = edge_index[0].astype(jnp.int32).reshape(NW, NB, B)
    dst3 = edge_index[1].astype(jnp.int32).reshape(NW, NB, B)
    xl, xr, xe = _proj(x, W_l, W_r)
    u = _gather_add(xl, xr, src3, dst3)
    alpha2d, gmax = _alpha(u, edge_attr, W_e, att)
    alpha3 = alpha2d.reshape(NW, NB, B)
    gmax16 = jnp.broadcast_to(gmax.reshape(()), (16,))
    zeros = jnp.zeros((NP, PW), jnp.float32)
    parts = _scatter(xe, src3, dst3, alpha3, gmax16, zeros)
    return _finalize(parts)


# speedup vs baseline: 1.0036x; 1.0036x over previous
import functools

import jax
import jax.numpy as jnp
from jax import lax
from jax.experimental import pallas as pl
from jax.experimental.pallas import tpu as pltpu
from jax.experimental.pallas import tpu_sc as plsc

N = 10000
E = 320000
D = 128
DE = 11
NC = 2
NS = 16
NW = NC * NS
EPW = E // NW
B = 80
NB = EPW // B
SB = 25
PW = 144
NP = 10240
RPT = NP // NS

_mesh = plsc.VectorSubcoreMesh(core_axis_name="c", subcore_axis_name="s")
_sc_params = pltpu.CompilerParams(needs_layout_passes=False,
                                  use_tc_tiling_on_sc=False)


def _proj_body(x_ref, wl_ref, wr_ref, xl_ref, xr_ref, xe_ref):
    x = x_ref[...]
    xl = jnp.dot(x, wl_ref[...], preferred_element_type=jnp.float32)
    xl_ref[...] = xl
    xr_ref[...] = jnp.dot(x, wr_ref[...], preferred_element_type=jnp.float32)
    xe_ref[...] = jnp.concatenate(
        [xl, jnp.ones((N, PW - D), jnp.float32)], axis=1)


def _proj(x, W_l, W_r):
    return pl.pallas_call(
        _proj_body,
        out_shape=(
            jax.ShapeDtypeStruct((N, D), jnp.float32),
            jax.ShapeDtypeStruct((N, D), jnp.float32),
            jax.ShapeDtypeStruct((N, PW), jnp.float32),
        ),
    )(x, W_l, W_r)


@functools.partial(
    pl.kernel,
    out_type=jax.ShapeDtypeStruct((E, D), jnp.float32),
    mesh=_mesh,
    scratch_types=[
        pltpu.VMEM((NB, B), jnp.int32),
        pltpu.VMEM((NB, B), jnp.int32),
        [pltpu.VMEM((B, D), jnp.float32)] * 3,
        [pltpu.VMEM((B, D), jnp.float32)] * 3,
        [pltpu.VMEM((B, D), jnp.float32)] * 3,
        [pltpu.SemaphoreType.DMA] * 3,
        [pltpu.SemaphoreType.DMA] * 3,
        [pltpu.SemaphoreType.DMA] * 3,
    ],
    compiler_params=_sc_params,
)
def _gather_add(xl_hbm, xr_hbm, src_hbm, dst_hbm, u_hbm,
                si_v, di_v, gl, gr, ob, gsl, gsr, ss):
    wid = lax.axis_index("s") * NC + lax.axis_index("c")
    pltpu.sync_copy(src_hbm.at[wid], si_v)
    pltpu.sync_copy(dst_hbm.at[wid], di_v)
    base = wid * EPW

    def fire(blk, k):
        pltpu.async_copy(xl_hbm.at[si_v.at[blk]], gl[k], gsl[k])
        pltpu.async_copy(xr_hbm.at[di_v.at[blk]], gr[k], gsr[k])

    def wait_g(blk, k):
        pltpu.make_async_copy(xl_hbm.at[si_v.at[blk]], gl[k], gsl[k]).wait()
        pltpu.make_async_copy(xr_hbm.at[di_v.at[blk]], gr[k], gsr[k]).wait()

    def compute(k):
        @plsc.parallel_loop(0, B, unroll=4)
        def _row(j):
            for cch in range(D // 16):
                sl = pl.ds(cch * 16, 16)
                ob[k][j, sl] = gl[k][j, sl] + gr[k][j, sl]

    def udst(blk):
        return u_hbm.at[pl.ds(base + blk * B, B)]

    def wait_s(blk, k):
        pltpu.make_async_copy(ob[k], udst(blk), ss[k]).wait()

    for k in range(3):
        fire(k, k)

    @pl.loop(0, NB - 2, step=3)
    def _trip(g):
        for k in range(3):
            b = g + k
            wait_g(b, k)

            @pl.when(g > 0)
            def _():
                wait_s(b - 3, k)

            compute(k)
            pltpu.async_copy(ob[k], udst(b), ss[k])
            if k < 2:
                fire(b + 3, k)
            else:
                @pl.when(g < NB - 5)
                def _():
                    fire(b + 3, k)

    for k, b in ((0, NB - 2), (1, NB - 1)):
        wait_g(b, k)
        wait_s(b - 3, k)
        compute(k)
        pltpu.async_copy(ob[k], udst(b), ss[k])
    wait_s(NB - 2, 0)
    wait_s(NB - 1, 1)
    wait_s(NB - 3, 2)


EB = 6400
GRID_A = E // EB


def _alpha_body(u_ref, ea_ref, we_ref, att_ref, alpha_ref, gmax_ref):
    e = lax.dot_general(ea_ref[...], we_ref[...], (((0,), (0,)), ((), ())),
                        preferred_element_type=jnp.float32)
    m = u_ref[...] + e
    m = jnp.where(m > 0, m, 0.2 * m)
    a = jnp.dot(m, att_ref[...], preferred_element_type=jnp.float32)[:, 0]
    alpha_ref[...] = a.reshape(1, EB // 128, 128)
    bm = jnp.max(a)

    @pl.when(pl.program_id(0) == 0)
    def _():
        gmax_ref[...] = jnp.full((1, 16), bm)

    @pl.when(pl.program_id(0) != 0)
    def _():
        gmax_ref[...] = jnp.maximum(gmax_ref[...], bm)


def _alpha(u, edge_attr, W_e, att):
    return pl.pallas_call(
        _alpha_body,
        grid=(GRID_A,),
        in_specs=[
            pl.BlockSpec((EB, D), lambda i: (i, 0)),
            pl.BlockSpec((DE, EB), lambda i: (0, i)),
            pl.BlockSpec((DE, D), lambda i: (0, 0)),
            pl.BlockSpec((D, 1), lambda i: (0, 0)),
        ],
        out_specs=[
            pl.BlockSpec((1, EB // 128, 128), lambda i: (i, 0, 0)),
            pl.BlockSpec((1, 16), lambda i: (0, 0)),
        ],
        out_shape=[
            jax.ShapeDtypeStruct((GRID_A, EB // 128, 128), jnp.float32),
            jax.ShapeDtypeStruct((1, 16), jnp.float32),
        ],
    )(u, edge_attr.T, W_e, att.reshape(D, 1))


@functools.partial(
    pl.kernel,
    out_type=jax.ShapeDtypeStruct((NC, NP, PW), jnp.float32),
    mesh=_mesh,
    scratch_types=[
        pltpu.VMEM((SB, B), jnp.int32),
        pltpu.VMEM((SB, B), jnp.int32),
        pltpu.VMEM((SB, B), jnp.float32),
        pltpu.VMEM((B, PW), jnp.float32),
        pltpu.VMEM((B, PW), jnp.float32),
        pltpu.VMEM((16,), jnp.float32),
        pltpu.VMEM_SHARED((NP, PW), jnp.float32),
        pltpu.SemaphoreType.DMA,
        pltpu.SemaphoreType.DMA,
        pltpu.SemaphoreType.DMA,
        pltpu.SemaphoreType.DMA,
    ],
    compiler_params=_sc_params,
)
def _scatter(xe_hbm, src_hbm, dst_hbm, alpha_hbm, gmax_hbm, zeros_hbm, part_hbm,
             si_v, di_v, al_v, gl0, gl1, gm_v, acc_sh, gs0, gs1, ss0, ss1):
    cid = lax.axis_index("c")
    sid = lax.axis_index("s")
    wid = sid * NC + cid
    pltpu.sync_copy(gmax_hbm, gm_v)
    pltpu.sync_copy(zeros_hbm.at[pl.ds(sid * RPT, RPT)],
                    acc_sh.at[pl.ds(sid * RPT, RPT)])
    plsc.subcore_barrier()

    def mult(blk, buf):
        blk16 = lax.broadcast(blk, (16,))

        @plsc.parallel_loop(0, B, unroll=4)
        def _row(j):
            j16 = lax.broadcast(j, (16,))
            eaj = plsc.load_gather(al_v, [blk16, j16])
            for cch in range(PW // 16):
                sl = pl.ds(cch * 16, 16)
                buf[j, sl] = buf[j, sl] * eaj

    def fire_g(blk, buf, sem):
        pltpu.async_copy(xe_hbm.at[si_v.at[blk]], buf, sem)

    def wait_g(blk, buf, sem):
        pltpu.make_async_copy(xe_hbm.at[si_v.at[blk]], buf, sem).wait()

    def fire_s(blk, buf, sem):
        pltpu.async_copy(buf, acc_sh.at[di_v.at[blk]], sem, add=True)

    def wait_s(buf, sem):
        pltpu.make_async_copy(buf, acc_sh.at[di_v.at[0]], sem).wait()

    @pl.loop(0, NB // SB)
    def _sblk(sb):
        @pl.when(sb > 0)
        def _():
            wait_s(gl1, ss1)

        pltpu.sync_copy(src_hbm.at[wid].at[pl.ds(sb * SB, SB)], si_v)
        pltpu.sync_copy(dst_hbm.at[wid].at[pl.ds(sb * SB, SB)], di_v)
        pltpu.sync_copy(alpha_hbm.at[wid].at[pl.ds(sb * SB, SB)], al_v)

        @pl.loop(0, SB)
        def _ea(r):
            for k in range(B // 16):
                sl = pl.ds(k * 16, 16)
                al_v[r, sl] = jnp.exp(al_v[r, sl] - gm_v[...])

        fire_g(0, gl0, gs0)

        @pl.loop(0, SB - 1, step=2)
        def _pair(g):
            @pl.when(g > 0)
            def _():
                wait_s(gl1, ss1)

            fire_g(g + 1, gl1, gs1)
            wait_g(g, gl0, gs0)
            mult(g, gl0)
            fire_s(g, gl0, ss0)
            wait_g(g + 1, gl1, gs1)
            mult(g + 1, gl1)
            fire_s(g + 1, gl1, ss1)
            wait_s(gl0, ss0)

            @pl.when(g + 2 < SB)
            def _():
                fire_g(g + 2, gl0, gs0)

        wait_g(SB - 1, gl0, gs0)
        mult(SB - 1, gl0)
        fire_s(SB - 1, gl0, ss0)
        wait_s(gl0, ss0)

    wait_s(gl1, ss1)
    plsc.subcore_barrier()
    st = sid * RPT
    pltpu.sync_copy(acc_sh.at[pl.ds(st, RPT)],
                    part_hbm.at[cid].at[pl.ds(st, RPT)])


FRB = 40
NF = NP // NW


@functools.partial(
    pl.kernel,
    out_type=jax.ShapeDtypeStruct((N, D), jnp.float32),
    mesh=_mesh,
    scratch_types=[
        pltpu.VMEM((FRB, PW), jnp.float32),
        pltpu.VMEM((FRB, PW), jnp.float32),
        pltpu.VMEM((FRB, D), jnp.float32),
        pltpu.SemaphoreType.DMA,
        pltpu.SemaphoreType.DMA,
    ],
    compiler_params=_sc_params,
)
def _finalize(part_hbm, out_hbm, p0_v, p1_v, o_v, s0, s1):
    wid = lax.axis_index("s") * NC + lax.axis_index("c")
    base = wid * NF

    @pl.loop(0, NF // FRB)
    def _blk(b):
        r0 = base + b * FRB

        @pl.when(r0 < N)
        def _():
            c0 = pltpu.async_copy(part_hbm.at[0].at[pl.ds(r0, FRB)], p0_v, s0)
            c1 = pltpu.async_copy(part_hbm.at[1].at[pl.ds(r0, FRB)], p1_v, s1)
            c0.wait()
            c1.wait()

            @plsc.parallel_loop(0, FRB, unroll=4)
            def _row(j):
                d16 = (p0_v[j, pl.ds(D, 16)] + p1_v[j, pl.ds(D, 16)]) + 1e-16
                for cch in range(D // 16):
                    sl = pl.ds(cch * 16, 16)
                    o_v[j, sl] = (p0_v[j, sl] + p1_v[j, sl]) / d16

            pltpu.sync_copy(o_v, out_hbm.at[pl.ds(r0, FRB)])


def kernel(x, edge_index, edge_attr, W_l, W_r, W_e, att):
    src3 = edge_index[0].astype(jnp.int32).reshape(NW, NB, B)
    dst3 = edge_index[1].astype(jnp.int32).reshape(NW, NB, B)
    xl, xr, xe = _proj(x, W_l, W_r)
    u = _gather_add(xl, xr, src3, dst3)
    alpha2d, gmax = _alpha(u, edge_attr, W_e, att)
    alpha3 = alpha2d.reshape(NW, NB, B)
    gmax16 = gmax.reshape(16)
    zeros = jnp.zeros((NP, PW), jnp.float32)
    parts = _scatter(xe, src3, dst3, alpha3, gmax16, zeros)
    return _finalize(parts)
